# baseline (speedup 1.0000x reference)
import dataclasses
import functools

import jax
import jax.numpy as jnp
from jax import lax
from jax.experimental import pallas as pl
from jax.experimental.pallas import tpu as pltpu
from jax.experimental.pallas import tpu_sc as plsc

N = 10000
E = 160000
D = 256
HALF = 128
N_LAYERS = 4

NC = 2
NS = 16
LANES = 16

CHUNK = 128
EP = 163840
EDGES_PER_SUB = EP // NS
NCHUNK = EDGES_PER_SUB // CHUNK
SLICE = 640
NROWS = N + 8
NP = NS * SLICE


def _zero_spmem_slices(sid, buf, shared):
    for k in range(SLICE // CHUNK):
        off = sid * SLICE + k * CHUNK

        @pl.when(off < N)
        def _():
            pltpu.sync_copy(buf, shared.at[pl.ds(off, CHUNK)])


def _dump_spmem_slices(cid, sid, shared, out_ref):
    for k in range(SLICE // CHUNK):
        off = sid * SLICE + k * CHUNK

        @pl.when(off < N)
        def _():
            sl = pl.ds(off, CHUNK)
            pltpu.sync_copy(shared.at[sl], out_ref.at[cid].at[sl])


def _sc_agg_body(h_ref, src_ref, de_ref, agg_ref,
                 idx_s2, de0, de1, rows0, rows1, sem0, sem1, semd0, semd1,
                 agg_sh):
    cid = lax.axis_index("c")
    sid = lax.axis_index("s")
    zvec = jnp.zeros((LANES,), jnp.float32)

    @pl.loop(0, CHUNK)
    def _(r):
        for j in range(HALF // LANES):
            rows0[r, pl.ds(j * LANES, LANES)] = zvec

    _zero_spmem_slices(sid, rows0, agg_sh)

    pltpu.sync_copy(src_ref.at[sid], idx_s2)

    coff = cid * N

    @pl.loop(0, NCHUNK)
    def _(i):
        for j in range(CHUNK // LANES):
            sl = pl.ds(j * LANES, LANES)
            idx_s2[i, sl] = idx_s2[i, sl] + coff

    plsc.subcore_barrier()

    my_de = de_ref.at[sid]

    def _issue(i, buf, sem, de, semd):
        pltpu.async_copy(h_ref.at[idx_s2.at[i]], buf, sem)
        pltpu.async_copy(my_de.at[i], de, semd)

    def _wait(buf, sem, de, semd):
        pltpu.make_async_copy(h_ref.at[idx_s2.at[0]], buf, sem).wait()
        pltpu.make_async_copy(my_de.at[0], de, semd).wait()

    def _scale_scatter(buf, de):
        @pl.loop(0, CHUNK // LANES)
        def _(g):
            evec = lax.bitcast_convert_type(de[1, pl.ds(g * LANES, LANES)],
                                            jnp.float32)
            for k in range(LANES):
                s = evec[k]
                r = g * LANES + k
                for j in range(HALF // LANES):
                    sl = pl.ds(j * LANES, LANES)
                    buf[r, sl] = buf[r, sl] * s

        pltpu.sync_copy(buf, agg_sh.at[de.at[0]], add=True)

    _issue(0, rows0, sem0, de0, semd0)

    @pl.loop(0, (NCHUNK - 2) // 2)
    def _(p):
        i0 = 2 * p
        _issue(i0 + 1, rows1, sem1, de1, semd1)
        _wait(rows0, sem0, de0, semd0)
        _scale_scatter(rows0, de0)
        _issue(i0 + 2, rows0, sem0, de0, semd0)
        _wait(rows1, sem1, de1, semd1)
        _scale_scatter(rows1, de1)

    _issue(NCHUNK - 1, rows1, sem1, de1, semd1)
    _wait(rows0, sem0, de0, semd0)
    _scale_scatter(rows0, de0)
    _wait(rows1, sem1, de1, semd1)
    _scale_scatter(rows1, de1)

    plsc.subcore_barrier()
    _dump_spmem_slices(cid, sid, agg_sh, agg_ref)


def _sc_deg_body(dst_ref, deg_ref, idxb, hist, red, out1, hist_sh):
    cid = lax.axis_index("c")
    sid = lax.axis_index("s")
    zvec = jnp.zeros((LANES,), jnp.float32)
    ovec = jnp.full((LANES,), 1.0, jnp.float32)

    @pl.loop(0, NP // LANES)
    def _(j):
        hist[pl.ds(j * LANES, LANES)] = zvec

    e0 = sid * EDGES_PER_SUB

    @pl.loop(0, NCHUNK)
    def _(i):
        pltpu.sync_copy(dst_ref.at[pl.ds(e0 + i * CHUNK, CHUNK)], idxb)

        @pl.loop(0, CHUNK // LANES)
        def _(g):
            iv = idxb[pl.ds(g * LANES, LANES)]
            plsc.addupdate_scatter(hist, [iv], ovec)

    pltpu.sync_copy(hist, hist_sh.at[sid])
    plsc.subcore_barrier()

    pltpu.sync_copy(hist_sh.at[:, pl.ds(sid * SLICE, SLICE)], red)

    @pl.loop(0, SLICE // LANES)
    def _(j):
        acc = red[0, pl.ds(j * LANES, LANES)]
        for t in range(1, NS):
            acc = acc + red[t, pl.ds(j * LANES, LANES)]
        out1[pl.ds(j * LANES, LANES)] = acc

    pltpu.sync_copy(out1, deg_ref.at[cid].at[pl.ds(sid * SLICE, SLICE)])


def _sc_compiler_params():
    cp = pltpu.CompilerParams()
    if "needs_layout_passes" in pltpu.CompilerParams.__dataclass_fields__:
        cp = dataclasses.replace(cp, needs_layout_passes=False)
    return cp


@functools.lru_cache(maxsize=None)
def _make_sc_agg():
    mesh = plsc.VectorSubcoreMesh(core_axis_name="c", subcore_axis_name="s")
    return pl.kernel(
        _sc_agg_body,
        out_type=[jax.ShapeDtypeStruct((NC, N, HALF), jnp.float32)],
        mesh=mesh,
        scratch_types=[
            pltpu.VMEM((NCHUNK, CHUNK), jnp.int32),
            pltpu.VMEM((2, CHUNK), jnp.int32),
            pltpu.VMEM((2, CHUNK), jnp.int32),
            pltpu.VMEM((CHUNK, HALF), jnp.float32),
            pltpu.VMEM((CHUNK, HALF), jnp.float32),
            pltpu.SemaphoreType.DMA,
            pltpu.SemaphoreType.DMA,
            pltpu.SemaphoreType.DMA,
            pltpu.SemaphoreType.DMA,
            pltpu.VMEM_SHARED((NROWS, HALF), jnp.float32),
        ],
    )


@functools.lru_cache(maxsize=None)
def _make_sc_deg():
    mesh = plsc.VectorSubcoreMesh(core_axis_name="c", subcore_axis_name="s")
    return pl.kernel(
        _sc_deg_body,
        out_type=[jax.ShapeDtypeStruct((NC, NP), jnp.float32)],
        mesh=mesh,
        compiler_params=_sc_compiler_params(),
        scratch_types=[
            pltpu.VMEM((CHUNK,), jnp.int32),
            pltpu.VMEM((NP,), jnp.float32),
            pltpu.VMEM((NS, SLICE), jnp.float32),
            pltpu.VMEM((SLICE,), jnp.float32),
            pltpu.VMEM_SHARED((NS, NP), jnp.float32),
        ],
    )


BLK = 1000


def _tc_layer_body(last, h_ref, agg_ref, deg_ref, ws_ref, wn_ref, b_ref, o_ref):
    h = jnp.concatenate([h_ref[0], h_ref[1]], axis=1)
    a = jnp.concatenate([agg_ref[0], agg_ref[1]], axis=1)
    inv = 1.0 / jnp.maximum(deg_ref[...], 1.0)
    out = (jnp.dot(h, ws_ref[...], preferred_element_type=jnp.float32)
           + jnp.dot(a * inv, wn_ref[...], preferred_element_type=jnp.float32)
           + b_ref[...])
    if not last:
        out = jnp.maximum(out, 0.0)
        o_ref[0] = out[:, :HALF]
        o_ref[1] = out[:, HALF:]
    else:
        o_ref[...] = out


def _make_tc_layer(last):
    if last:
        out_shape = jax.ShapeDtypeStruct((N, D), jnp.float32)
        out_spec = pl.BlockSpec((BLK, D), lambda i: (i, 0))
    else:
        out_shape = jax.ShapeDtypeStruct((NC, N, HALF), jnp.float32)
        out_spec = pl.BlockSpec((NC, BLK, HALF), lambda i: (0, i, 0))
    return pl.pallas_call(
        functools.partial(_tc_layer_body, last),
        grid=(N // BLK,),
        in_specs=[
            pl.BlockSpec((NC, BLK, HALF), lambda i: (0, i, 0)),
            pl.BlockSpec((NC, BLK, HALF), lambda i: (0, i, 0)),
            pl.BlockSpec((BLK, 1), lambda i: (i, 0)),
            pl.BlockSpec((D, D), lambda i: (0, 0)),
            pl.BlockSpec((D, D), lambda i: (0, 0)),
            pl.BlockSpec((1, D), lambda i: (0, 0)),
        ],
        out_specs=out_spec,
        out_shape=out_shape,
    )


_tc_layer = _make_tc_layer(False)
_tc_layer_last = _make_tc_layer(True)


def kernel(x, edge_index, pos, W_self, W_neigh, b):
    src = edge_index[0]
    dst = edge_index[1]
    h3 = jnp.stack([x[:, :HALF], x[:, HALF:]], axis=0)

    pad = EP - E
    srcp = jnp.concatenate([src, jnp.zeros((pad,), jnp.int32)])
    dstp = jnp.concatenate([dst, jnp.full((pad,), N, jnp.int32)])
    ewp = jnp.concatenate([pos, jnp.zeros((pad,), jnp.float32)])

    (deg_full,) = _make_sc_deg()(dstp)
    deg = deg_full[0, :N][:, None]

    src3 = srcp.reshape(NS, NCHUNK, CHUNK)
    ew_bits = jax.lax.bitcast_convert_type(ewp, jnp.int32)
    de4 = jnp.stack([dstp.reshape(NS, NCHUNK, CHUNK),
                     ew_bits.reshape(NS, NCHUNK, CHUNK)], axis=2)

    (agg,) = _make_sc_agg()(h3.reshape(NC * N, HALF), src3, de4)
    h3 = _tc_layer(h3, agg, deg, W_self[0], W_neigh[0], b[0].reshape(1, D))

    for l in range(1, N_LAYERS - 1):
        (agg,) = _make_sc_agg()(h3.reshape(NC * N, HALF), src3, de4)
        h3 = _tc_layer(h3, agg, deg, W_self[l], W_neigh[l], b[l].reshape(1, D))

    (agg,) = _make_sc_agg()(h3.reshape(NC * N, HALF), src3, de4)
    out = _tc_layer_last(h3, agg, deg, W_self[N_LAYERS - 1], W_neigh[N_LAYERS - 1],
                         b[N_LAYERS - 1].reshape(1, D))
    return out


# speedup vs baseline: 1.8665x; 1.8665x over previous
import dataclasses
import functools

import jax
import jax.numpy as jnp
from jax import lax
from jax.experimental import pallas as pl
from jax.experimental.pallas import tpu as pltpu
from jax.experimental.pallas import tpu_sc as plsc

N = 10000
E = 160000
D = 256
HALF = 128
N_LAYERS = 4

NC = 2
NS = 16
LANES = 16

CHUNK = 80
EDGES_PER_SUB = E // NS
NCHUNK = EDGES_PER_SUB // CHUNK
SLICE = 640
DCH = 128
EP = 163840
DEPS = EP // NS
DNCH = DEPS // DCH
NP = NS * SLICE


def _zero_spmem_slices(sid, buf, shared):
    for k in range(SLICE // CHUNK):
        off = sid * SLICE + k * CHUNK

        @pl.when(off < N)
        def _():
            pltpu.sync_copy(buf, shared.at[pl.ds(off, CHUNK)])


def _dump_spmem_slices(cid, sid, shared, out_ref):
    for k in range(SLICE // CHUNK):
        off = sid * SLICE + k * CHUNK

        @pl.when(off < N)
        def _():
            sl = pl.ds(off, CHUNK)
            pltpu.sync_copy(shared.at[sl], out_ref.at[cid].at[sl])


def _sc_agg_body(h_ref, src_ref, de_ref, agg_ref,
                 idx_s2, de0, de1, rows0, rows1, sem0, sem1, semd0, semd1,
                 agg_sh):
    cid = lax.axis_index("c")
    sid = lax.axis_index("s")
    zvec = jnp.zeros((LANES,), jnp.float32)

    @pl.loop(0, CHUNK)
    def _(r):
        for j in range(HALF // LANES):
            rows0[r, pl.ds(j * LANES, LANES)] = zvec

    _zero_spmem_slices(sid, rows0, agg_sh)

    pltpu.sync_copy(src_ref.at[sid], idx_s2)

    coff = cid * N

    @pl.loop(0, NCHUNK)
    def _(i):
        for j in range(CHUNK // LANES):
            sl = pl.ds(j * LANES, LANES)
            idx_s2[i, sl] = idx_s2[i, sl] + coff

    plsc.subcore_barrier()

    my_de = de_ref.at[sid]

    def _issue(i, buf, sem, de, semd):
        pltpu.async_copy(h_ref.at[idx_s2.at[i]], buf, sem)
        pltpu.async_copy(my_de.at[i], de, semd)

    def _wait(buf, sem, de, semd):
        pltpu.make_async_copy(h_ref.at[idx_s2.at[0]], buf, sem).wait()
        pltpu.make_async_copy(my_de.at[0], de, semd).wait()

    def _scale_scatter(buf, de):
        @pl.loop(0, CHUNK // LANES)
        def _(g):
            evec = lax.bitcast_convert_type(de[1, pl.ds(g * LANES, LANES)],
                                            jnp.float32)
            for k in range(LANES):
                s = evec[k]
                r = g * LANES + k
                for j in range(HALF // LANES):
                    sl = pl.ds(j * LANES, LANES)
                    buf[r, sl] = buf[r, sl] * s

        pltpu.sync_copy(buf, agg_sh.at[de.at[0]], add=True)

    _issue(0, rows0, sem0, de0, semd0)

    @pl.loop(0, NCHUNK // 2)
    def _(p):
        i0 = 2 * p
        _issue(i0 + 1, rows1, sem1, de1, semd1)
        _wait(rows0, sem0, de0, semd0)
        _scale_scatter(rows0, de0)
        _issue(i0 + 2, rows0, sem0, de0, semd0)
        _wait(rows1, sem1, de1, semd1)
        _scale_scatter(rows1, de1)

    _wait(rows0, sem0, de0, semd0)
    _scale_scatter(rows0, de0)

    plsc.subcore_barrier()
    _dump_spmem_slices(cid, sid, agg_sh, agg_ref)


def _sc_deg_body(dst_ref, deg_ref, idxb, hist, red, out1, hist_sh):
    cid = lax.axis_index("c")
    sid = lax.axis_index("s")
    zvec = jnp.zeros((LANES,), jnp.float32)
    ovec = jnp.full((LANES,), 1.0, jnp.float32)

    @pl.loop(0, NP // LANES)
    def _(j):
        hist[pl.ds(j * LANES, LANES)] = zvec

    e0 = sid * DEPS

    @pl.loop(0, DNCH)
    def _(i):
        pltpu.sync_copy(dst_ref.at[pl.ds(e0 + i * DCH, DCH)], idxb)

        @pl.loop(0, DCH // LANES)
        def _(g):
            iv = idxb[pl.ds(g * LANES, LANES)]
            plsc.addupdate_scatter(hist, [iv], ovec)

    pltpu.sync_copy(hist, hist_sh.at[sid])
    plsc.subcore_barrier()

    pltpu.sync_copy(hist_sh.at[:, pl.ds(sid * SLICE, SLICE)], red)

    @pl.loop(0, SLICE // LANES)
    def _(j):
        acc = red[0, pl.ds(j * LANES, LANES)]
        for t in range(1, NS):
            acc = acc + red[t, pl.ds(j * LANES, LANES)]
        out1[pl.ds(j * LANES, LANES)] = acc

    pltpu.sync_copy(out1, deg_ref.at[cid].at[pl.ds(sid * SLICE, SLICE)])


def _sc_compiler_params():
    cp = pltpu.CompilerParams()
    if "needs_layout_passes" in pltpu.CompilerParams.__dataclass_fields__:
        cp = dataclasses.replace(cp, needs_layout_passes=False)
    return cp


@functools.lru_cache(maxsize=None)
def _make_sc_agg():
    mesh = plsc.VectorSubcoreMesh(core_axis_name="c", subcore_axis_name="s")
    return pl.kernel(
        _sc_agg_body,
        out_type=[jax.ShapeDtypeStruct((NC, N, HALF), jnp.float32)],
        mesh=mesh,
        scratch_types=[
            pltpu.VMEM((NCHUNK, CHUNK), jnp.int32),
            pltpu.VMEM((2, CHUNK), jnp.int32),
            pltpu.VMEM((2, CHUNK), jnp.int32),
            pltpu.VMEM((CHUNK, HALF), jnp.float32),
            pltpu.VMEM((CHUNK, HALF), jnp.float32),
            pltpu.SemaphoreType.DMA,
            pltpu.SemaphoreType.DMA,
            pltpu.SemaphoreType.DMA,
            pltpu.SemaphoreType.DMA,
            pltpu.VMEM_SHARED((N, HALF), jnp.float32),
        ],
    )


@functools.lru_cache(maxsize=None)
def _make_sc_deg():
    mesh = plsc.VectorSubcoreMesh(core_axis_name="c", subcore_axis_name="s")
    return pl.kernel(
        _sc_deg_body,
        out_type=[jax.ShapeDtypeStruct((NC, NP), jnp.float32)],
        mesh=mesh,
        compiler_params=_sc_compiler_params(),
        scratch_types=[
            pltpu.VMEM((DCH,), jnp.int32),
            pltpu.VMEM((NP,), jnp.float32),
            pltpu.VMEM((NS, SLICE), jnp.float32),
            pltpu.VMEM((SLICE,), jnp.float32),
            pltpu.VMEM_SHARED((NS, NP), jnp.float32),
        ],
    )


BLK = 1000


def _tc_layer_body(last, h_ref, agg_ref, deg_ref, ws_ref, wn_ref, b_ref, o_ref):
    h = jnp.concatenate([h_ref[0], h_ref[1]], axis=1)
    a = jnp.concatenate([agg_ref[0], agg_ref[1]], axis=1)
    inv = 1.0 / jnp.maximum(deg_ref[...], 1.0)
    out = (jnp.dot(h, ws_ref[...], preferred_element_type=jnp.float32)
           + jnp.dot(a * inv, wn_ref[...], preferred_element_type=jnp.float32)
           + b_ref[...])
    if not last:
        out = jnp.maximum(out, 0.0)
        o_ref[0] = out[:, :HALF]
        o_ref[1] = out[:, HALF:]
    else:
        o_ref[...] = out


def _make_tc_layer(last):
    if last:
        out_shape = jax.ShapeDtypeStruct((N, D), jnp.float32)
        out_spec = pl.BlockSpec((BLK, D), lambda i: (i, 0))
    else:
        out_shape = jax.ShapeDtypeStruct((NC, N, HALF), jnp.float32)
        out_spec = pl.BlockSpec((NC, BLK, HALF), lambda i: (0, i, 0))
    return pl.pallas_call(
        functools.partial(_tc_layer_body, last),
        grid=(N // BLK,),
        in_specs=[
            pl.BlockSpec((NC, BLK, HALF), lambda i: (0, i, 0)),
            pl.BlockSpec((NC, BLK, HALF), lambda i: (0, i, 0)),
            pl.BlockSpec((BLK, 1), lambda i: (i, 0)),
            pl.BlockSpec((D, D), lambda i: (0, 0)),
            pl.BlockSpec((D, D), lambda i: (0, 0)),
            pl.BlockSpec((1, D), lambda i: (0, 0)),
        ],
        out_specs=out_spec,
        out_shape=out_shape,
    )


_tc_layer = _make_tc_layer(False)
_tc_layer_last = _make_tc_layer(True)


def kernel(x, edge_index, pos, W_self, W_neigh, b):
    src = edge_index[0]
    dst = edge_index[1]
    h3 = jnp.stack([x[:, :HALF], x[:, HALF:]], axis=0)

    dstp = jnp.concatenate([dst, jnp.full((EP - E,), N, jnp.int32)])
    (deg_full,) = _make_sc_deg()(dstp)
    deg = deg_full[0, :N][:, None]

    src3 = src.reshape(NS, NCHUNK, CHUNK)
    ew_bits = jax.lax.bitcast_convert_type(pos, jnp.int32)
    de4 = jnp.stack([dst.reshape(NS, NCHUNK, CHUNK),
                     ew_bits.reshape(NS, NCHUNK, CHUNK)], axis=2)

    (agg,) = _make_sc_agg()(h3.reshape(NC * N, HALF), src3, de4)
    h3 = _tc_layer(h3, agg, deg, W_self[0], W_neigh[0], b[0].reshape(1, D))

    for l in range(1, N_LAYERS - 1):
        (agg,) = _make_sc_agg()(h3.reshape(NC * N, HALF), src3, de4)
        h3 = _tc_layer(h3, agg, deg, W_self[l], W_neigh[l], b[l].reshape(1, D))

    (agg,) = _make_sc_agg()(h3.reshape(NC * N, HALF), src3, de4)
    out = _tc_layer_last(h3, agg, deg, W_self[N_LAYERS - 1], W_neigh[N_LAYERS - 1],
                         b[N_LAYERS - 1].reshape(1, D))
    return out
